# baseline (speedup 1.0000x reference)
import functools

import jax
import jax.numpy as jnp
from jax.experimental import pallas as pl


def _vq_body(z_ref, cb_ref, zq_ref, idx_ref, loss_ref):
    b = pl.program_id(0)
    zb = z_ref[0]
    cb = cb_ref[...]
    K = cb.shape[0]
    N = zb.shape[1]

    cnorm = jnp.sum(cb * cb, axis=1)
    scores = jax.lax.dot_general(
        cb, zb, (((1,), (0,)), ((), ())),
        preferred_element_type=jnp.float32)
    d = cnorm[:, None] - 2.0 * scores

    dmin = jnp.min(d, axis=0)
    iota_k = jax.lax.broadcasted_iota(jnp.int32, (K, N), 0)
    idx = jnp.min(jnp.where(d == dmin[None, :], iota_k, K), axis=0)

    xnorm = jnp.sum(zb * zb, axis=0)
    loss_part = jnp.sum(dmin + xnorm)

    onehot = (iota_k == idx[None, :]).astype(jnp.float32)
    zq = jax.lax.dot_general(
        cb, onehot, (((0,), (0,)), ((), ())),
        preferred_element_type=jnp.float32)

    zq_ref[0] = zq
    idx_ref[0, 0] = idx

    loss_blk = jnp.reshape(loss_part, (1, 1))

    @pl.when(b == 0)
    def _init():
        loss_ref[...] = loss_blk

    @pl.when(b > 0)
    def _acc():
        loss_ref[...] += loss_blk


@functools.partial(jax.jit, static_argnames=())
def _vq(z3, codebook):
    B, C, N = z3.shape
    K = codebook.shape[0]
    zq, idx, loss = pl.pallas_call(
        _vq_body,
        grid=(B,),
        in_specs=[
            pl.BlockSpec((1, C, N), lambda b: (b, 0, 0)),
            pl.BlockSpec((K, C), lambda b: (0, 0)),
        ],
        out_specs=[
            pl.BlockSpec((1, C, N), lambda b: (b, 0, 0)),
            pl.BlockSpec((1, 1, N), lambda b: (b, 0, 0)),
            pl.BlockSpec((1, 1), lambda b: (0, 0)),
        ],
        out_shape=[
            jax.ShapeDtypeStruct((B, C, N), jnp.float32),
            jax.ShapeDtypeStruct((B, 1, N), jnp.int32),
            jax.ShapeDtypeStruct((1, 1), jnp.float32),
        ],
    )(z3, codebook)
    return zq, idx, loss


def kernel(z, embedding_weight):
    b, c, h, w = z.shape
    z3 = z.reshape(b, c, h * w)
    zq, idx, loss = _vq(z3, embedding_weight)
    z_q_out = zq.reshape(b, c, h, w)
    codebook_loss = loss[0, 0] * 1.25 / (b * c * h * w)
    indices_out = idx.reshape(b, 1, h, w)
    return (z_q_out, codebook_loss, indices_out)


# speedup vs baseline: 1.0456x; 1.0456x over previous
import functools

import jax
import jax.numpy as jnp
from jax.experimental import pallas as pl


def _vq_body(z_ref, cb_ref, zq_ref, idx_ref, loss_ref):
    b = pl.program_id(0)
    zb = z_ref[0]
    cb = cb_ref[...]
    K = cb.shape[0]
    N = zb.shape[1]

    cnorm = jnp.sum(cb * cb, axis=1)
    scores = jax.lax.dot_general(
        cb, zb, (((1,), (0,)), ((), ())),
        preferred_element_type=jnp.float32)
    d = cnorm[:, None] - 2.0 * scores

    dmin = jnp.min(d, axis=0)
    idx = jnp.argmin(d, axis=0).astype(jnp.int32)

    xnorm = jnp.sum(zb * zb, axis=0)
    loss_part = jnp.sum(dmin + xnorm)

    iota_k = jax.lax.broadcasted_iota(jnp.int32, (K, N), 0)
    onehot = (iota_k == idx[None, :]).astype(jnp.bfloat16)
    zq = jax.lax.dot_general(
        cb.astype(jnp.bfloat16), onehot, (((0,), (0,)), ((), ())),
        preferred_element_type=jnp.float32)

    zq_ref[0] = zq
    idx_ref[0, 0] = idx

    loss_blk = jnp.reshape(loss_part, (1, 1))

    @pl.when(b == 0)
    def _init():
        loss_ref[...] = loss_blk

    @pl.when(b > 0)
    def _acc():
        loss_ref[...] += loss_blk


@functools.partial(jax.jit, static_argnames=())
def _vq(z3, codebook):
    B, C, N = z3.shape
    K = codebook.shape[0]
    zq, idx, loss = pl.pallas_call(
        _vq_body,
        grid=(B,),
        in_specs=[
            pl.BlockSpec((1, C, N), lambda b: (b, 0, 0)),
            pl.BlockSpec((K, C), lambda b: (0, 0)),
        ],
        out_specs=[
            pl.BlockSpec((1, C, N), lambda b: (b, 0, 0)),
            pl.BlockSpec((1, 1, N), lambda b: (b, 0, 0)),
            pl.BlockSpec((1, 1), lambda b: (0, 0)),
        ],
        out_shape=[
            jax.ShapeDtypeStruct((B, C, N), jnp.float32),
            jax.ShapeDtypeStruct((B, 1, N), jnp.int32),
            jax.ShapeDtypeStruct((1, 1), jnp.float32),
        ],
    )(z3, codebook)
    return zq, idx, loss


def kernel(z, embedding_weight):
    b, c, h, w = z.shape
    z3 = z.reshape(b, c, h * w)
    zq, idx, loss = _vq(z3, embedding_weight)
    z_q_out = zq.reshape(b, c, h, w)
    codebook_loss = loss[0, 0] * 1.25 / (b * c * h * w)
    indices_out = idx.reshape(b, 1, h, w)
    return (z_q_out, codebook_loss, indices_out)
